# baseline (speedup 1.0000x reference)
import functools

import jax
import jax.numpy as jnp
from jax import lax
from jax.experimental import pallas as pl
from jax.experimental.pallas import tpu as pltpu
from jax.experimental.pallas import tpu_sc as plsc

VOCAB = 128
EMBED_DIM = 64
PAIR_DIM = 2 * EMBED_DIM

_info = plsc.get_sparse_core_info()
_NC, _NS = _info.num_cores, _info.num_subcores
_NW = _NC * _NS


@functools.partial(jax.jit, static_argnames=("npairs",))
def _lookup(pids, pair_table, *, npairs):
    chunk = 400
    per_w = npairs // _NW
    nchunks = per_w // chunk
    mesh = plsc.VectorSubcoreMesh(core_axis_name="c", subcore_axis_name="s")

    @functools.partial(
        pl.kernel,
        mesh=mesh,
        out_type=jax.ShapeDtypeStruct((npairs, PAIR_DIM), jnp.float32),
        scratch_types=[
            pltpu.VMEM((2, chunk), jnp.int32),
            pltpu.VMEM((2, chunk, PAIR_DIM), jnp.float32),
            pltpu.SemaphoreType.DMA,
            pltpu.SemaphoreType.DMA,
            pltpu.SemaphoreType.DMA,
            pltpu.SemaphoreType.DMA,
            pltpu.SemaphoreType.DMA,
            pltpu.SemaphoreType.DMA,
        ],
        compiler_params=pltpu.CompilerParams(use_tc_tiling_on_sc=False),
    )
    def k(pid_hbm, ptab_hbm, out_hbm, pid_v, rows_v,
          isem0, isem1, gsem0, gsem1, wsem0, wsem1):
        wid = lax.axis_index("s") * _NC + lax.axis_index("c")
        base = wid * per_w
        isems = (isem0, isem1)
        gsems = (gsem0, gsem1)
        wsems = (wsem0, wsem1)

        for par in range(2):
            pltpu.async_copy(
                pid_hbm.at[pl.ds(base + par * chunk, chunk)],
                pid_v.at[par],
                isems[par],
            )

        def super_chunk(c2, _):
            for par in range(2):
                c = 2 * c2 + par
                pltpu.make_async_copy(
                    pid_hbm.at[pl.ds(base, chunk)], pid_v.at[par], isems[par]
                ).wait()

                @pl.when(c2 > 0)
                def _drain():
                    pltpu.make_async_copy(
                        rows_v.at[par], out_hbm.at[pl.ds(base, chunk)],
                        wsems[par],
                    ).wait()

                pltpu.async_copy(
                    ptab_hbm.at[pid_v.at[par]], rows_v.at[par], gsems[par]
                ).wait()

                @pl.when(c < nchunks - 2)
                def _prefetch():
                    pltpu.async_copy(
                        pid_hbm.at[pl.ds(base + (c + 2) * chunk, chunk)],
                        pid_v.at[par],
                        isems[par],
                    )

                pltpu.async_copy(
                    rows_v.at[par],
                    out_hbm.at[pl.ds(base + c * chunk, chunk)],
                    wsems[par],
                )
            return ()

        lax.fori_loop(0, nchunks // 2, super_chunk, ())

        for par in range(2):
            pltpu.make_async_copy(
                rows_v.at[par], out_hbm.at[pl.ds(base, chunk)], wsems[par]
            ).wait()

    return k(pids, pair_table)


def kernel(indices, table):
    B, L = indices.shape
    npairs = B * L // 2
    idx2 = indices.reshape(npairs, 2).astype(jnp.int32)
    pids = idx2[:, 0] * VOCAB + idx2[:, 1]
    left = jnp.repeat(table, VOCAB, axis=0)
    right = jnp.tile(table, (VOCAB, 1))
    pair_table = jnp.concatenate([left, right], axis=1)
    out = _lookup(pids, pair_table, npairs=npairs)
    return out.reshape(B, L, EMBED_DIM)


# speedup vs baseline: 1.5270x; 1.5270x over previous
import functools

import jax
import jax.numpy as jnp
from jax import lax
from jax.experimental import pallas as pl
from jax.experimental.pallas import tpu as pltpu
from jax.experimental.pallas import tpu_sc as plsc

EMBED_DIM = 64

_info = plsc.get_sparse_core_info()
_NC, _NS = _info.num_cores, _info.num_subcores
_NW = _NC * _NS


@functools.partial(jax.jit, static_argnames=("batch", "length"))
def _lookup(idx2d, table, *, batch, length):
    chunk_b = 4
    chunk = chunk_b * length
    per_w_b = batch // _NW
    nchunks = per_w_b // chunk_b
    mesh = plsc.VectorSubcoreMesh(core_axis_name="c", subcore_axis_name="s")

    @functools.partial(
        pl.kernel,
        mesh=mesh,
        out_type=jax.ShapeDtypeStruct((batch, length, EMBED_DIM), jnp.float32),
        scratch_types=[
            pltpu.VMEM((2, chunk_b, length), jnp.int32),
            pltpu.VMEM((2, chunk_b, length, EMBED_DIM), jnp.float32),
            pltpu.VMEM_SHARED((128, EMBED_DIM), jnp.float32),
            pltpu.SemaphoreType.DMA,
            pltpu.SemaphoreType.DMA,
            pltpu.SemaphoreType.DMA,
            pltpu.SemaphoreType.DMA,
            pltpu.SemaphoreType.DMA,
            pltpu.SemaphoreType.DMA,
        ],
        compiler_params=pltpu.CompilerParams(use_tc_tiling_on_sc=False),
    )
    def k(idx_hbm, table_hbm, out_hbm, idx_v, rows_v, table_sh,
          isem0, isem1, gsem0, gsem1, wsem0, wsem1):
        sid = lax.axis_index("s")
        wid = sid * _NC + lax.axis_index("c")
        b_base = wid * per_w_b
        isems = (isem0, isem1)
        gsems = (gsem0, gsem1)
        wsems = (wsem0, wsem1)

        @pl.when(sid == 0)
        def _stage_table():
            pltpu.sync_copy(table_hbm, table_sh)

        plsc.subcore_barrier()

        for par in range(2):
            pltpu.async_copy(
                idx_hbm.at[pl.ds(b_base + par * chunk_b, chunk_b)],
                idx_v.at[par],
                isems[par],
            )

        def super_chunk(c2, _):
            for par in range(2):
                c = 2 * c2 + par
                pltpu.make_async_copy(
                    idx_hbm.at[pl.ds(b_base, chunk_b)], idx_v.at[par],
                    isems[par],
                ).wait()
                @pl.when(c2 > 0)
                def _drain():
                    pltpu.make_async_copy(
                        rows_v.at[par], out_hbm.at[pl.ds(b_base, chunk_b)],
                        wsems[par],
                    ).wait()

                handles = [
                    pltpu.async_copy(
                        table_sh.at[idx_v.at[par, cb]],
                        rows_v.at[par, cb],
                        gsems[par],
                    )
                    for cb in range(chunk_b)
                ]
                for h in handles:
                    h.wait()

                @pl.when(c < nchunks - 2)
                def _prefetch():
                    pltpu.async_copy(
                        idx_hbm.at[pl.ds(b_base + (c + 2) * chunk_b, chunk_b)],
                        idx_v.at[par],
                        isems[par],
                    )

                pltpu.async_copy(
                    rows_v.at[par],
                    out_hbm.at[pl.ds(b_base + c * chunk_b, chunk_b)],
                    wsems[par],
                )
            return ()

        lax.fori_loop(0, nchunks // 2, super_chunk, ())

        for par in range(2):
            pltpu.make_async_copy(
                rows_v.at[par], out_hbm.at[pl.ds(b_base, chunk_b)], wsems[par]
            ).wait()

    return k(idx2d, table)


def kernel(indices, table):
    B, L = indices.shape
    return _lookup(indices.astype(jnp.int32), table, batch=B, length=L)


# speedup vs baseline: 1.5309x; 1.0026x over previous
import functools

import jax
import jax.numpy as jnp
from jax import lax
from jax.experimental import pallas as pl
from jax.experimental.pallas import tpu as pltpu
from jax.experimental.pallas import tpu_sc as plsc

EMBED_DIM = 64

_info = plsc.get_sparse_core_info()
_NC, _NS = _info.num_cores, _info.num_subcores
_NW = _NC * _NS


@functools.partial(jax.jit, static_argnames=("batch", "length"))
def _lookup(idx2d, table, *, batch, length):
    chunk_b = 2
    chunk = chunk_b * length
    per_w_b = batch // _NW
    nchunks = per_w_b // chunk_b
    mesh = plsc.VectorSubcoreMesh(core_axis_name="c", subcore_axis_name="s")

    @functools.partial(
        pl.kernel,
        mesh=mesh,
        out_type=jax.ShapeDtypeStruct((batch, length, EMBED_DIM), jnp.float32),
        scratch_types=[
            pltpu.VMEM((2, chunk_b, length), jnp.int32),
            pltpu.VMEM((2, chunk_b, length, EMBED_DIM), jnp.float32),
            pltpu.VMEM_SHARED((128, EMBED_DIM), jnp.float32),
            pltpu.SemaphoreType.DMA,
            pltpu.SemaphoreType.DMA,
            pltpu.SemaphoreType.DMA,
            pltpu.SemaphoreType.DMA,
            pltpu.SemaphoreType.DMA,
            pltpu.SemaphoreType.DMA,
        ],
        compiler_params=pltpu.CompilerParams(use_tc_tiling_on_sc=False),
    )
    def k(idx_hbm, table_hbm, out_hbm, idx_v, rows_v, table_sh,
          isem0, isem1, gsem0, gsem1, wsem0, wsem1):
        sid = lax.axis_index("s")
        wid = sid * _NC + lax.axis_index("c")
        b_base = wid * per_w_b
        isems = (isem0, isem1)
        gsems = (gsem0, gsem1)
        wsems = (wsem0, wsem1)

        @pl.when(sid == 0)
        def _stage_table():
            pltpu.sync_copy(table_hbm, table_sh)

        plsc.subcore_barrier()

        for par in range(2):
            pltpu.async_copy(
                idx_hbm.at[pl.ds(b_base + par * chunk_b, chunk_b)],
                idx_v.at[par],
                isems[par],
            )

        def super_chunk(c2, _):
            for par in range(2):
                c = 2 * c2 + par
                pltpu.make_async_copy(
                    idx_hbm.at[pl.ds(b_base, chunk_b)], idx_v.at[par],
                    isems[par],
                ).wait()
                @pl.when(c2 > 0)
                def _drain():
                    pltpu.make_async_copy(
                        rows_v.at[par], out_hbm.at[pl.ds(b_base, chunk_b)],
                        wsems[par],
                    ).wait()

                handles = [
                    pltpu.async_copy(
                        table_sh.at[idx_v.at[par, cb]],
                        rows_v.at[par, cb],
                        gsems[par],
                    )
                    for cb in range(chunk_b)
                ]
                for h in handles:
                    h.wait()

                @pl.when(c < nchunks - 2)
                def _prefetch():
                    pltpu.async_copy(
                        idx_hbm.at[pl.ds(b_base + (c + 2) * chunk_b, chunk_b)],
                        idx_v.at[par],
                        isems[par],
                    )

                pltpu.async_copy(
                    rows_v.at[par],
                    out_hbm.at[pl.ds(b_base + c * chunk_b, chunk_b)],
                    wsems[par],
                )
            return ()

        lax.fori_loop(0, nchunks // 2, super_chunk, ())

        for par in range(2):
            pltpu.make_async_copy(
                rows_v.at[par], out_hbm.at[pl.ds(b_base, chunk_b)], wsems[par]
            ).wait()

    return k(idx2d, table)


def kernel(indices, table):
    B, L = indices.shape
    return _lookup(indices.astype(jnp.int32), table, batch=B, length=L)
